# baseline (speedup 1.0000x reference)
import functools

import jax
import jax.numpy as jnp
from jax import lax
from jax.experimental import pallas as pl
from jax.experimental.pallas import tpu as pltpu
from jax.experimental.pallas import tpu_sc as plsc

V, D, P = 1000, 64, 512
B, L = 4096, 200
NC, NS = 2, 16
NW = NC * NS
N = B * L
TPW = N // NW
G = 128
NG = TPW // G

_mesh = plsc.VectorSubcoreMesh(core_axis_name="c", subcore_axis_name="s")


@functools.partial(
    pl.kernel,
    mesh=_mesh,
    compiler_params=pltpu.CompilerParams(use_tc_tiling_on_sc=False),
    out_type=jax.ShapeDtypeStruct((N, D), jnp.float32),
    scratch_types=[
        pltpu.VMEM((NG, G), jnp.int32),
        pltpu.VMEM((NG, G), jnp.int32),
        pltpu.VMEM((G, D), jnp.float32),
        pltpu.VMEM((G, D), jnp.float32),
        pltpu.SemaphoreType.DMA,
        pltpu.SemaphoreType.DMA,
    ],
)
def _emb(x_hbm, p_hbm, tok_hbm, pos_hbm, out_hbm, xi, pi, tr, pr, s1, s2):
    wid = lax.axis_index("s") * NC + lax.axis_index("c")
    base = wid * TPW
    pltpu.sync_copy(x_hbm.at[wid], xi)
    pltpu.sync_copy(p_hbm.at[wid], pi)

    def tile(g, carry):
        ct = pltpu.async_copy(tok_hbm.at[xi.at[g]], tr, s1)
        ct.wait()
        cp = pltpu.async_copy(pos_hbm.at[pi.at[g]], tr, s2, add=True)
        cp.wait()
        pltpu.sync_copy(tr, out_hbm.at[pl.ds(base + g * G, G)])
        return carry

    lax.fori_loop(0, NG, tile, 0)


def kernel(x, pos, token_table, pos_table):
    xf = x.reshape(NW, NG, G).astype(jnp.int32)
    pf = pos.reshape(NW, NG, G).astype(jnp.int32)
    out = _emb(xf, pf, token_table, pos_table)
    return out.reshape(B, L, D)


# speedup vs baseline: 1.4390x; 1.4390x over previous
import functools

import jax
import jax.numpy as jnp
from jax import lax
from jax.experimental import pallas as pl
from jax.experimental.pallas import tpu as pltpu
from jax.experimental.pallas import tpu_sc as plsc

V, D, P = 1000, 64, 512
B, L = 4096, 200
NC, NS = 2, 16
NW = NC * NS
N = B * L
TPW = N // NW
G = 128
NG = TPW // G

_mesh = plsc.VectorSubcoreMesh(core_axis_name="c", subcore_axis_name="s")


@functools.partial(
    pl.kernel,
    mesh=_mesh,
    compiler_params=pltpu.CompilerParams(use_tc_tiling_on_sc=False),
    out_type=jax.ShapeDtypeStruct((N, D), jnp.float32),
    scratch_types=[
        pltpu.VMEM((NG, G), jnp.int32),
        pltpu.VMEM((NG, G), jnp.int32),
        pltpu.VMEM((G, D), jnp.float32),
        pltpu.VMEM_SHARED((V, D), jnp.float32),
        pltpu.VMEM_SHARED((P, D), jnp.float32),
        pltpu.SemaphoreType.DMA,
        pltpu.SemaphoreType.DMA,
    ],
)
def _emb(x_hbm, p_hbm, tok_hbm, pos_hbm, out_hbm, xi, pi, tr, stok, spos, s1, s2):
    sid = lax.axis_index("s")
    wid = sid * NC + lax.axis_index("c")
    base = wid * TPW

    @pl.when(sid == 0)
    def _stage():
        pltpu.sync_copy(tok_hbm, stok)
        pltpu.sync_copy(pos_hbm, spos)

    pltpu.sync_copy(x_hbm.at[wid], xi)
    pltpu.sync_copy(p_hbm.at[wid], pi)
    plsc.subcore_barrier()

    def tile(g, carry):
        ct = pltpu.async_copy(stok.at[xi.at[g]], tr, s1)
        ct.wait()
        cp = pltpu.async_copy(spos.at[pi.at[g]], tr, s2, add=True)
        cp.wait()
        pltpu.sync_copy(tr, out_hbm.at[pl.ds(base + g * G, G)])
        return carry

    lax.fori_loop(0, NG, tile, 0)


def kernel(x, pos, token_table, pos_table):
    xf = x.reshape(NW, NG, G).astype(jnp.int32)
    pf = pos.reshape(NW, NG, G).astype(jnp.int32)
    out = _emb(xf, pf, token_table, pos_table)
    return out.reshape(B, L, D)


# speedup vs baseline: 1.6056x; 1.1158x over previous
import functools

import jax
import jax.numpy as jnp
from jax import lax
from jax.experimental import pallas as pl
from jax.experimental.pallas import tpu as pltpu
from jax.experimental.pallas import tpu_sc as plsc

V, D, P = 1000, 64, 512
B, L = 4096, 200
NC, NS = 2, 16
NW = NC * NS
N = B * L
TPW = N // NW
G = 128
NG = TPW // G
NBUF = 6
DADD = 2
KOUT = 4

_mesh = plsc.VectorSubcoreMesh(core_axis_name="c", subcore_axis_name="s")


@functools.partial(
    pl.kernel,
    mesh=_mesh,
    compiler_params=pltpu.CompilerParams(use_tc_tiling_on_sc=False),
    out_type=jax.ShapeDtypeStruct((N, D), jnp.float32),
    scratch_types=[
        pltpu.VMEM((NG, G), jnp.int32),
        pltpu.VMEM((NG, G), jnp.int32),
        pltpu.VMEM((NBUF, G, D), jnp.float32),
        pltpu.VMEM_SHARED((V, D), jnp.float32),
        pltpu.VMEM_SHARED((P, D), jnp.float32),
        pltpu.SemaphoreType.DMA((NBUF,)),
        pltpu.SemaphoreType.DMA((NBUF,)),
        pltpu.SemaphoreType.DMA((NBUF,)),
    ],
)
def _emb(x_hbm, p_hbm, tok_hbm, pos_hbm, out_hbm, xi, pi, tr, stok, spos,
         st, sa, so):
    sid = lax.axis_index("s")
    wid = sid * NC + lax.axis_index("c")
    base = wid * TPW

    @pl.when(sid == 0)
    def _stage():
        pltpu.sync_copy(tok_hbm, stok)
        pltpu.sync_copy(pos_hbm, spos)

    pltpu.sync_copy(x_hbm.at[wid], xi)
    pltpu.sync_copy(p_hbm.at[wid], pi)
    plsc.subcore_barrier()

    def slot(s, b, carry):
        g_tok = s
        g_add = s - DADD
        g_out = s - KOUT
        b_add = (b - DADD) % NBUF
        b_out = (b - KOUT) % NBUF

        @pl.when((g_tok >= NBUF) & (g_tok < NG))
        def _reuse():
            pltpu.make_async_copy(tr.at[b], out_hbm.at[pl.ds(0, G)], so.at[b]).wait()

        @pl.when(g_tok < NG)
        def _tok():
            pltpu.async_copy(stok.at[xi.at[g_tok]], tr.at[b], st.at[b])

        @pl.when((0 <= g_add) & (g_add < NG))
        def _add():
            pltpu.make_async_copy(stok.at[xi.at[g_add]], tr.at[b_add],
                                  st.at[b_add]).wait()
            pltpu.async_copy(spos.at[pi.at[g_add]], tr.at[b_add], sa.at[b_add],
                             add=True)

        @pl.when(0 <= g_out)
        def _out():
            pltpu.make_async_copy(spos.at[pi.at[g_out]], tr.at[b_out],
                                  sa.at[b_out]).wait()
            pltpu.async_copy(tr.at[b_out], out_hbm.at[pl.ds(base + g_out * G, G)],
                             so.at[b_out])

        return carry

    def block(bo, carry):
        s0 = bo * NBUF
        for b in range(NBUF):
            slot(s0 + b, b, carry)
        return carry

    lax.fori_loop(0, (NG + KOUT) // NBUF, block, 0)

    for b in range(NBUF):
        pltpu.make_async_copy(tr.at[b], out_hbm.at[pl.ds(0, G)], so.at[b]).wait()


def kernel(x, pos, token_table, pos_table):
    xf = x.reshape(NW, NG, G).astype(jnp.int32)
    pf = pos.reshape(NW, NG, G).astype(jnp.int32)
    out = _emb(xf, pf, token_table, pos_table)
    return out.reshape(B, L, D)
